# baseline (speedup 1.0000x reference)
import functools

import jax
import jax.numpy as jnp
from jax import lax
from jax.experimental import pallas as pl
from jax.experimental.pallas import tpu as pltpu
from jax.experimental.pallas import tpu_sc as plsc

VOCAB = 1000
EMBED = 16
B = 1024
L = 50

VPAD = 1024
NW = 32
VSTRIPE = VPAD // NW
NTR = VSTRIPE // 8
TRS = VOCAB // 8
STRIPE_ELEMS = VSTRIPE * B
OUT_CHUNK = NTR * 8 * B
L_STRIDE = TRS * 8 * B


def _table_body(w_ref, e_ref, b_ref, out_ref):
    out_ref[...] = (
        lax.dot_general(
            w_ref[...], e_ref[...],
            dimension_numbers=(((0,), (1,)), ((), ())),
            preferred_element_type=jnp.float32,
        )
        + b_ref[...]
    )


def _table_T(emb_table, lm_w, lm_b):
    wpad = jnp.zeros((EMBED, VPAD), jnp.float32).at[:, :VOCAB].set(lm_w)
    epad = jnp.zeros((VPAD, EMBED), jnp.float32).at[:VOCAB].set(emb_table)
    bpad = jnp.zeros((VPAD, 1), jnp.float32).at[:VOCAB, 0].set(lm_b)
    return pl.pallas_call(
        _table_body,
        out_shape=jax.ShapeDtypeStruct((VPAD, VPAD), jnp.float32),
    )(wpad, epad, bpad)


_sc_mesh = plsc.VectorSubcoreMesh(
    core_axis_name="c", subcore_axis_name="s", num_cores=2, num_subcores=16
)


@functools.partial(
    pl.kernel,
    out_type=jax.ShapeDtypeStruct((L * TRS * 8 * B,), jnp.float32),
    mesh=_sc_mesh,
    compiler_params=pltpu.CompilerParams(
        use_tc_tiling_on_sc=False, needs_layout_passes=False
    ),
    scratch_types=[
        pltpu.VMEM((STRIPE_ELEMS,), jnp.float32),
        pltpu.VMEM((OUT_CHUNK,), jnp.float32),
        pltpu.VMEM((OUT_CHUNK,), jnp.float32),
        pltpu.VMEM((B,), jnp.int32),
        pltpu.VMEM((B,), jnp.int32),
        pltpu.SemaphoreType.DMA,
        pltpu.SemaphoreType.DMA,
        pltpu.SemaphoreType.DMA,
        pltpu.SemaphoreType.DMA,
        pltpu.SemaphoreType.DMA,
    ],
)
def _sc_gather(tableT_hbm, idsT_hbm, out_hbm, stripe, out0, out1, ids0, ids1,
               sem_s, sem_i0, sem_i1, sem_o0, sem_o1):
    wid = lax.axis_index("s") * 2 + lax.axis_index("c")
    v0 = jnp.minimum(wid * VSTRIPE, VOCAB - VSTRIPE)

    pltpu.async_copy(
        tableT_hbm.at[pl.ds(v0 * B, STRIPE_ELEMS)], stripe, sem_s
    ).wait()

    def compute(ids_buf, out_buf):
        @plsc.parallel_loop(0, B // 16, 1, unroll=8)
        def g_body(g):
            idx16 = ids_buf[pl.ds(g * 16, 16)]
            base = (g // 8) * 1024 + (g % 8) * 16
            for v in range(VSTRIPE):
                val = plsc.load_gather(stripe.at[pl.ds(v * B, B)], [idx16])
                off = base + (v // 8) * 8192 + (v % 8) * 128
                out_buf[pl.ds(off, 16)] = val

    def ids_start(l, buf, sem):
        pltpu.async_copy(idsT_hbm.at[pl.ds(l * B, B)], buf, sem)

    def ids_drain(buf, sem):
        pltpu.make_async_copy(idsT_hbm.at[pl.ds(0, B)], buf, sem).wait()

    def out_start(l, buf, sem):
        pltpu.async_copy(
            buf, out_hbm.at[pl.ds(l * L_STRIDE + v0 * B, OUT_CHUNK)], sem
        )

    def out_drain(buf, sem):
        pltpu.make_async_copy(buf, out_hbm.at[pl.ds(0, OUT_CHUNK)], sem).wait()

    pltpu.async_copy(idsT_hbm.at[pl.ds(0, B)], ids0, sem_i0).wait()
    ids_start(1, ids1, sem_i1)
    compute(ids0, out0)
    out_start(0, out0, sem_o0)
    ids_start(2, ids0, sem_i0)
    ids_drain(ids1, sem_i1)
    compute(ids1, out1)
    out_start(1, out1, sem_o1)
    ids_start(3, ids1, sem_i1)

    def body(i, carry):
        l0 = 2 * i + 2
        l1 = l0 + 1
        ids_drain(ids0, sem_i0)
        out_drain(out0, sem_o0)
        compute(ids0, out0)
        out_start(l0, out0, sem_o0)
        ids_start(l0 + 2, ids0, sem_i0)
        ids_drain(ids1, sem_i1)
        out_drain(out1, sem_o1)
        compute(ids1, out1)
        out_start(l1, out1, sem_o1)
        ids_start(l1 + 2, ids1, sem_i1)
        return carry

    lax.fori_loop(0, L // 2 - 2, body, 0)
    ids_drain(ids0, sem_i0)
    out_drain(out0, sem_o0)
    compute(ids0, out0)
    out_start(L - 2, out0, sem_o0)
    ids_drain(ids1, sem_i1)
    out_drain(out1, sem_o1)
    compute(ids1, out1)
    out_start(L - 1, out1, sem_o1)
    out_drain(out0, sem_o0)
    out_drain(out1, sem_o1)


def kernel(input_ids, emb_table, lm_w, lm_b):
    table_T = _table_T(emb_table, lm_w, lm_b)
    ids_flat = input_ids.astype(jnp.int32).T.reshape(-1)
    out_flat = _sc_gather(table_T.reshape(-1), ids_flat)
    out = (
        out_flat.reshape(L, TRS, 8, 8, 128)
        .transpose(0, 1, 3, 2, 4)
        .reshape(L, VOCAB, B)
        .transpose(2, 0, 1)
    )
    return out


# speedup vs baseline: 1.0057x; 1.0057x over previous
import functools

import jax
import jax.numpy as jnp
from jax import lax
from jax.experimental import pallas as pl
from jax.experimental.pallas import tpu as pltpu
from jax.experimental.pallas import tpu_sc as plsc

VOCAB = 1000
EMBED = 16
B = 1024
L = 50

VPAD = 1024
NW = 32
VSTRIPE = VPAD // NW
NTR = VSTRIPE // 8
TRS = VOCAB // 8
STRIPE_ELEMS = VSTRIPE * B
OUT_CHUNK = NTR * 8 * B
L_STRIDE = TRS * 8 * B


def _table_body(w_ref, e_ref, b_ref, out_ref):
    out_ref[...] = (
        lax.dot_general(
            w_ref[...], e_ref[...],
            dimension_numbers=(((0,), (1,)), ((), ())),
            preferred_element_type=jnp.float32,
        )
        + b_ref[...]
    )


def _table_T(emb_table, lm_w, lm_b):
    wpad = jnp.zeros((EMBED, VPAD), jnp.float32).at[:, :VOCAB].set(lm_w)
    epad = jnp.zeros((VPAD, EMBED), jnp.float32).at[:VOCAB].set(emb_table)
    bpad = jnp.zeros((VPAD, 1), jnp.float32).at[:VOCAB, 0].set(lm_b)
    return pl.pallas_call(
        _table_body,
        out_shape=jax.ShapeDtypeStruct((VPAD, VPAD), jnp.float32),
    )(wpad, epad, bpad)


_sc_mesh = plsc.VectorSubcoreMesh(
    core_axis_name="c", subcore_axis_name="s", num_cores=2, num_subcores=16
)


@functools.partial(
    pl.kernel,
    out_type=jax.ShapeDtypeStruct((L * TRS * 8 * B,), jnp.float32),
    mesh=_sc_mesh,
    compiler_params=pltpu.CompilerParams(
        use_tc_tiling_on_sc=False, needs_layout_passes=False
    ),
    scratch_types=[
        pltpu.VMEM((STRIPE_ELEMS,), jnp.float32),
        pltpu.VMEM((OUT_CHUNK,), jnp.float32),
        pltpu.VMEM((OUT_CHUNK,), jnp.float32),
        pltpu.VMEM((B,), jnp.int32),
        pltpu.VMEM((B,), jnp.int32),
        pltpu.SemaphoreType.DMA,
        pltpu.SemaphoreType.DMA,
        pltpu.SemaphoreType.DMA,
        pltpu.SemaphoreType.DMA,
        pltpu.SemaphoreType.DMA,
    ],
)
def _sc_gather(tableT_hbm, idsT_hbm, out_hbm, stripe, out0, out1, ids0, ids1,
               sem_s, sem_i0, sem_i1, sem_o0, sem_o1):
    wid = lax.axis_index("s") * 2 + lax.axis_index("c")
    v0 = jnp.minimum(wid * VSTRIPE, VOCAB - VSTRIPE)

    pltpu.async_copy(
        tableT_hbm.at[pl.ds(v0 * B, STRIPE_ELEMS)], stripe, sem_s
    ).wait()

    def compute(ids_buf, out_buf):
        @plsc.parallel_loop(0, B // 16, 1, unroll=4)
        def g_body(g):
            idx16 = ids_buf[pl.ds(g * 16, 16)]
            base = (g // 8) * 1024 + (g % 8) * 16
            for v in range(VSTRIPE):
                val = plsc.load_gather(stripe.at[pl.ds(v * B, B)], [idx16])
                off = base + (v // 8) * 8192 + (v % 8) * 128
                out_buf[pl.ds(off, 16)] = val

    def ids_start(l, buf, sem):
        pltpu.async_copy(idsT_hbm.at[pl.ds(l * B, B)], buf, sem)

    def ids_drain(buf, sem):
        pltpu.make_async_copy(idsT_hbm.at[pl.ds(0, B)], buf, sem).wait()

    def out_start(l, buf, sem):
        pltpu.async_copy(
            buf, out_hbm.at[pl.ds(l * L_STRIDE + v0 * B, OUT_CHUNK)], sem
        )

    def out_drain(buf, sem):
        pltpu.make_async_copy(buf, out_hbm.at[pl.ds(0, OUT_CHUNK)], sem).wait()

    pltpu.async_copy(idsT_hbm.at[pl.ds(0, B)], ids0, sem_i0).wait()
    ids_start(1, ids1, sem_i1)
    compute(ids0, out0)
    out_start(0, out0, sem_o0)
    ids_start(2, ids0, sem_i0)
    ids_drain(ids1, sem_i1)
    compute(ids1, out1)
    out_start(1, out1, sem_o1)
    ids_start(3, ids1, sem_i1)

    def body(i, carry):
        l0 = 2 * i + 2
        l1 = l0 + 1
        ids_drain(ids0, sem_i0)
        out_drain(out0, sem_o0)
        compute(ids0, out0)
        out_start(l0, out0, sem_o0)
        ids_start(l0 + 2, ids0, sem_i0)
        ids_drain(ids1, sem_i1)
        out_drain(out1, sem_o1)
        compute(ids1, out1)
        out_start(l1, out1, sem_o1)
        ids_start(l1 + 2, ids1, sem_i1)
        return carry

    lax.fori_loop(0, L // 2 - 2, body, 0)
    ids_drain(ids0, sem_i0)
    out_drain(out0, sem_o0)
    compute(ids0, out0)
    out_start(L - 2, out0, sem_o0)
    ids_drain(ids1, sem_i1)
    out_drain(out1, sem_o1)
    compute(ids1, out1)
    out_start(L - 1, out1, sem_o1)
    out_drain(out0, sem_o0)
    out_drain(out1, sem_o1)


def kernel(input_ids, emb_table, lm_w, lm_b):
    table_T = _table_T(emb_table, lm_w, lm_b)
    ids_flat = input_ids.astype(jnp.int32).T.reshape(-1)
    out_flat = _sc_gather(table_T.reshape(-1), ids_flat)
    out = (
        out_flat.reshape(L, TRS, 8, 8, 128)
        .transpose(0, 1, 3, 2, 4)
        .reshape(L, VOCAB, B)
        .transpose(2, 0, 1)
    )
    return out


# speedup vs baseline: 1.0490x; 1.0430x over previous
import functools

import jax
import jax.numpy as jnp
from jax import lax
from jax.experimental import pallas as pl
from jax.experimental.pallas import tpu as pltpu
from jax.experimental.pallas import tpu_sc as plsc

VOCAB = 1000
EMBED = 16
B = 1024
L = 50

VPAD = 1024
NW = 32
VSTRIPE = VPAD // NW
NTR = VSTRIPE // 8
TRS = VOCAB // 8
STRIPE_ELEMS = VSTRIPE * B
OUT_CHUNK = NTR * 8 * B
L_STRIDE = TRS * 8 * B


def _table_body(w_ref, e_ref, b_ref, out_ref):
    out_ref[0:VOCAB, 0:VOCAB] = (
        lax.dot_general(
            w_ref[...], e_ref[...],
            dimension_numbers=(((0,), (1,)), ((), ())),
            preferred_element_type=jnp.float32,
        )
        + b_ref[...]
    )


def _table_T(emb_table, lm_w, lm_b):
    return pl.pallas_call(
        _table_body,
        out_shape=jax.ShapeDtypeStruct((VPAD, VPAD), jnp.float32),
    )(lm_w, emb_table, lm_b.reshape(VOCAB, 1))


_sc_mesh = plsc.VectorSubcoreMesh(
    core_axis_name="c", subcore_axis_name="s", num_cores=2, num_subcores=16
)


@functools.partial(
    pl.kernel,
    out_type=jax.ShapeDtypeStruct((L * TRS * 8 * B,), jnp.float32),
    mesh=_sc_mesh,
    compiler_params=pltpu.CompilerParams(
        use_tc_tiling_on_sc=False, needs_layout_passes=False
    ),
    scratch_types=[
        pltpu.VMEM((STRIPE_ELEMS,), jnp.float32),
        pltpu.VMEM((OUT_CHUNK,), jnp.float32),
        pltpu.VMEM((OUT_CHUNK,), jnp.float32),
        pltpu.VMEM((B,), jnp.int32),
        pltpu.VMEM((B,), jnp.int32),
        pltpu.SemaphoreType.DMA,
        pltpu.SemaphoreType.DMA,
        pltpu.SemaphoreType.DMA,
        pltpu.SemaphoreType.DMA,
        pltpu.SemaphoreType.DMA,
    ],
)
def _sc_gather(tableT_hbm, idsT_hbm, out_hbm, stripe, out0, out1, ids0, ids1,
               sem_s, sem_i0, sem_i1, sem_o0, sem_o1):
    wid = lax.axis_index("s") * 2 + lax.axis_index("c")
    v0 = jnp.minimum(wid * VSTRIPE, VOCAB - VSTRIPE)

    pltpu.async_copy(
        tableT_hbm.at[pl.ds(v0 * B, STRIPE_ELEMS)], stripe, sem_s
    ).wait()

    def compute(ids_buf, out_buf):
        @plsc.parallel_loop(0, B // 16, 1, unroll=4)
        def g_body(g):
            idx16 = ids_buf[pl.ds(g * 16, 16)]
            ivb = ((idx16 >> 7) << 10) + (idx16 & 127)
            base = (g // 8) * 1024 + (g % 8) * 16
            for v in range(VSTRIPE):
                soff = (v // 8) * 8192 + (v % 8) * 128
                val = plsc.load_gather(stripe.at[pl.ds(soff, 7 * 1024 + 128)], [ivb])
                off = base + (v // 8) * 8192 + (v % 8) * 128
                out_buf[pl.ds(off, 16)] = val

    def ids_start(l, buf, sem):
        pltpu.async_copy(idsT_hbm.at[pl.ds(l * B, B)], buf, sem)

    def ids_drain(buf, sem):
        pltpu.make_async_copy(idsT_hbm.at[pl.ds(0, B)], buf, sem).wait()

    def out_start(l, buf, sem):
        pltpu.async_copy(
            buf, out_hbm.at[pl.ds(l * L_STRIDE + v0 * B, OUT_CHUNK)], sem
        )

    def out_drain(buf, sem):
        pltpu.make_async_copy(buf, out_hbm.at[pl.ds(0, OUT_CHUNK)], sem).wait()

    pltpu.async_copy(idsT_hbm.at[pl.ds(0, B)], ids0, sem_i0).wait()
    ids_start(1, ids1, sem_i1)
    compute(ids0, out0)
    out_start(0, out0, sem_o0)
    ids_start(2, ids0, sem_i0)
    ids_drain(ids1, sem_i1)
    compute(ids1, out1)
    out_start(1, out1, sem_o1)
    ids_start(3, ids1, sem_i1)

    def body(i, carry):
        l0 = 2 * i + 2
        l1 = l0 + 1
        ids_drain(ids0, sem_i0)
        out_drain(out0, sem_o0)
        compute(ids0, out0)
        out_start(l0, out0, sem_o0)
        ids_start(l0 + 2, ids0, sem_i0)
        ids_drain(ids1, sem_i1)
        out_drain(out1, sem_o1)
        compute(ids1, out1)
        out_start(l1, out1, sem_o1)
        ids_start(l1 + 2, ids1, sem_i1)
        return carry

    lax.fori_loop(0, L // 2 - 2, body, 0)
    ids_drain(ids0, sem_i0)
    out_drain(out0, sem_o0)
    compute(ids0, out0)
    out_start(L - 2, out0, sem_o0)
    ids_drain(ids1, sem_i1)
    out_drain(out1, sem_o1)
    compute(ids1, out1)
    out_start(L - 1, out1, sem_o1)
    out_drain(out0, sem_o0)
    out_drain(out1, sem_o1)


def kernel(input_ids, emb_table, lm_w, lm_b):
    table_T = _table_T(emb_table, lm_w, lm_b)
    table_bytes = (
        table_T.reshape(VPAD // 8, 8, VPAD // 128, 128)
        .transpose(0, 2, 1, 3)
        .reshape(-1)
    )
    ids_flat = input_ids.astype(jnp.int32).T.reshape(-1)
    out_flat = _sc_gather(table_bytes, ids_flat)
    out = (
        out_flat.reshape(L, TRS, 8, 8, 128)
        .transpose(0, 1, 3, 2, 4)
        .reshape(L, VOCAB, B)
        .transpose(2, 0, 1)
    )
    return out
